# baseline (speedup 1.0000x reference)
import functools

import jax
import jax.numpy as jnp
from jax import lax
from jax.experimental import pallas as pl
from jax.experimental.pallas import tpu as pltpu
from jax.experimental.pallas import tpu_sc as plsc

MAX_REL = 512
D = 64
LQ = 32
LK = 8192
NT = 2 * MAX_REL + 1
VW = 640
TBASE = 384
SREAL = 1024 - TBASE
SW = SREAL + 32
BW = 384
SPW = 2048
TAIL = LK - VW
NSP = 2
NBC = (TAIL - NSP * SPW) // BW
NLANE = 16
NSPLAT = (VW - (MAX_REL + LQ)) // NLANE


def kernel(length_q, length_k, embeddings_table):
    del length_q, length_k

    table_t = embeddings_table.T
    last_row = embeddings_table[NT - 1:NT, :]

    info = plsc.get_sparse_core_info()
    nc = info.num_cores

    mesh = plsc.VectorSubcoreMesh(core_axis_name="c", subcore_axis_name="s")

    @functools.partial(
        pl.kernel,
        mesh=mesh,
        out_type=jax.ShapeDtypeStruct((LQ, D, LK), jnp.float32),
        compiler_params=pltpu.CompilerParams(needs_layout_passes=False),
        scratch_types=[
            pltpu.VMEM((D, SW), jnp.float32),
            pltpu.VMEM((D, VW), jnp.float32),
            pltpu.VMEM((D, BW), jnp.float32),
            pltpu.VMEM((1, D), jnp.float32),
            pltpu.VMEM_SHARED((D, SPW), jnp.float32),
            pltpu.SemaphoreType.DMA,
            pltpu.SemaphoreType.DMA,
            pltpu.SemaphoreType.DMA,
        ],
    )
    def sc_kernel(lr_hbm, tt_hbm, out_hbm, stage_v, var_v, bc_v, lv_v,
                  sp_big, sem_a, sem_b, sem_out):
        c = lax.axis_index("c")
        s = lax.axis_index("s")
        w = s * nc + c

        big = pltpu.async_copy(tt_hbm.at[:, pl.ds(TBASE, SREAL)],
                               stage_v.at[:, pl.ds(0, SREAL)], sem_a)
        small = pltpu.async_copy(lr_hbm, lv_v, sem_b)
        small.wait()

        zvec = jnp.full((NLANE,), 0, jnp.int32)

        def bc_body(d, carry):
            vsp = plsc.load_gather(
                lv_v, [zvec, jnp.full((NLANE,), d, jnp.int32)])
            for j in range(BW // NLANE):
                bc_v[d, pl.ds(NLANE * j, NLANE)] = vsp
            return carry
        lax.fori_loop(0, D, bc_body, 0)

        for j in range(SPW // BW):
            @pl.when(s == j)
            def _(j=j):
                pltpu.sync_copy(bc_v, sp_big.at[:, pl.ds(j * BW, BW)])

        @pl.when(s == SPW // BW)
        def _():
            pltpu.sync_copy(bc_v.at[:, pl.ds(0, SPW - (SPW // BW) * BW)],
                            sp_big.at[:, pl.ds((SPW // BW) * BW,
                                               SPW - (SPW // BW) * BW)])
        plsc.subcore_barrier()

        pending = []
        for i in range(NSP):
            pending.append(pltpu.async_copy(
                sp_big, out_hbm.at[w, :, pl.ds(VW + i * SPW, SPW)], sem_out))
        for i in range(NBC):
            pending.append(pltpu.async_copy(
                bc_v, out_hbm.at[w, :, pl.ds(VW + NSP * SPW + i * BW, BW)],
                sem_out))

        big.wait()
        def pad_body(d, carry):
            vsp = bc_v[d, pl.ds(0, NLANE)]
            stage_v[d, pl.ds(SREAL, NLANE)] = vsp
            stage_v[d, pl.ds(SREAL + NLANE, NLANE)] = vsp
            return carry
        lax.fori_loop(0, D, pad_body, 0)

        ngather = VW // NLANE - NSPLAT
        starts = [jnp.minimum(MAX_REL - TBASE + NLANE * b - w, SREAL)
                  for b in range(ngather)]
        lane_iota = lax.iota(jnp.int32, NLANE)

        def var_body(d, carry):
            dvec = jnp.full((NLANE,), d, jnp.int32)
            for b in range(ngather):
                idx = lane_iota + starts[b]
                var_v[d, pl.ds(NLANE * b, NLANE)] = \
                    plsc.load_gather(stage_v, [dvec, idx])
            vsp = bc_v[d, pl.ds(0, NLANE)]
            for b in range(ngather, VW // NLANE):
                var_v[d, pl.ds(NLANE * b, NLANE)] = vsp
            return carry
        lax.fori_loop(0, D, var_body, 0)

        pending.append(pltpu.async_copy(
            var_v, out_hbm.at[w, :, pl.ds(0, VW)], sem_out))
        for p in pending:
            p.wait()

    out_t = sc_kernel(last_row, table_t)
    return jnp.transpose(out_t, (0, 2, 1))


# speedup vs baseline: 1.0624x; 1.0624x over previous
import functools

import jax
import jax.numpy as jnp
from jax import lax
from jax.experimental import pallas as pl
from jax.experimental.pallas import tpu as pltpu
from jax.experimental.pallas import tpu_sc as plsc

MAX_REL = 512
D = 64
LQ = 32
LK = 8192
NT = 2 * MAX_REL + 1
VW = 640
TBASE = 384
SREAL = 1024 - TBASE
SW = SREAL + 32
BW = 512
TAIL = LK - VW
NBC = TAIL // BW
REM = TAIL - NBC * BW
NLANE = 16
NSPLAT = (VW - (MAX_REL + LQ)) // NLANE


def kernel(length_q, length_k, embeddings_table):
    del length_q, length_k

    table_t = embeddings_table.T
    last_row = embeddings_table[NT - 1:NT, :]

    info = plsc.get_sparse_core_info()
    nc = info.num_cores

    mesh = plsc.VectorSubcoreMesh(core_axis_name="c", subcore_axis_name="s")

    @functools.partial(
        pl.kernel,
        mesh=mesh,
        out_type=jax.ShapeDtypeStruct((LQ, D, LK), jnp.float32),
        compiler_params=pltpu.CompilerParams(needs_layout_passes=False),
        scratch_types=[
            pltpu.VMEM((D, SW), jnp.float32),
            pltpu.VMEM((D, VW), jnp.float32),
            pltpu.VMEM((D, BW), jnp.float32),
            pltpu.VMEM((1, D), jnp.float32),
            pltpu.VMEM_SHARED((D, BW), jnp.float32),
            pltpu.VMEM_SHARED((D, BW), jnp.float32),
            pltpu.SemaphoreType.DMA,
            pltpu.SemaphoreType.DMA,
            pltpu.SemaphoreType.DMA,
        ],
    )
    def sc_kernel(lr_hbm, tt_hbm, out_hbm, stage_v, var_v, bc_v, lv_v,
                  sp_bc, sp_bc2, sem_a, sem_b, sem_out):
        c = lax.axis_index("c")
        s = lax.axis_index("s")
        w = s * nc + c

        big = pltpu.async_copy(tt_hbm.at[:, pl.ds(TBASE, SREAL)],
                               stage_v.at[:, pl.ds(0, SREAL)], sem_a)
        small = pltpu.async_copy(lr_hbm, lv_v, sem_b)
        small.wait()

        zvec = jnp.full((NLANE,), 0, jnp.int32)

        def bc_body(d, carry):
            vsp = plsc.load_gather(
                lv_v, [zvec, jnp.full((NLANE,), d, jnp.int32)])
            for j in range(BW // NLANE):
                bc_v[d, pl.ds(NLANE * j, NLANE)] = vsp
            return carry
        lax.fori_loop(0, D, bc_body, 0)

        @pl.when(s == 0)
        def _():
            pltpu.sync_copy(bc_v, sp_bc)

        @pl.when(s == 1)
        def _():
            pltpu.sync_copy(bc_v, sp_bc2)
        plsc.subcore_barrier()

        pending = []
        srcs = [bc_v, sp_bc, sp_bc2]
        for i in range(NBC):
            pending.append(pltpu.async_copy(
                srcs[i % 3], out_hbm.at[w, :, pl.ds(VW + i * BW, BW)],
                sem_out))
        pending.append(pltpu.async_copy(
            bc_v.at[:, pl.ds(0, REM)],
            out_hbm.at[w, :, pl.ds(VW + NBC * BW, REM)], sem_out))

        big.wait()
        def pad_body(d, carry):
            vsp = bc_v[d, pl.ds(0, NLANE)]
            stage_v[d, pl.ds(SREAL, NLANE)] = vsp
            stage_v[d, pl.ds(SREAL + NLANE, NLANE)] = vsp
            return carry
        lax.fori_loop(0, D, pad_body, 0)

        ngather = VW // NLANE - NSPLAT
        starts = [jnp.minimum(MAX_REL - TBASE + NLANE * b - w, SREAL)
                  for b in range(ngather)]
        lane_iota = lax.iota(jnp.int32, NLANE)

        def var_body(d, carry):
            dvec = jnp.full((NLANE,), d, jnp.int32)
            for b in range(ngather):
                idx = lane_iota + starts[b]
                var_v[d, pl.ds(NLANE * b, NLANE)] = \
                    plsc.load_gather(stage_v, [dvec, idx])
            vsp = bc_v[d, pl.ds(0, NLANE)]
            for b in range(ngather, VW // NLANE):
                var_v[d, pl.ds(NLANE * b, NLANE)] = vsp
            return carry
        lax.fori_loop(0, D, var_body, 0)

        pending.append(pltpu.async_copy(
            var_v, out_hbm.at[w, :, pl.ds(0, VW)], sem_out))
        for p in pending:
            p.wait()

    out_t = sc_kernel(last_row, table_t)
    return jnp.transpose(out_t, (0, 2, 1))
